# Initial kernel scaffold; baseline (speedup 1.0000x reference)
import jax
import jax.numpy as jnp
from jax.experimental import pallas as pl


def kernel(atom_fea, nbr_fea, state_fea, atom_nbr_idx, node_atom_idx, W_a, b_a, W_b, b_b, W_s, b_s):
    raise NotImplementedError("write your pallas kernel here")


# speedup vs baseline: 1.8936x; 1.8936x over previous
import functools

import jax
import jax.numpy as jnp
from jax import lax
from jax.experimental import pallas as pl
from jax.experimental.pallas import tpu as pltpu
from jax.experimental.pallas import tpu_sc as plsc


def _softplus(x):
    return jnp.maximum(x, 0.0) + jnp.log1p(jnp.exp(-jnp.abs(x)))


def _sc_gather(table, idx_flat):
    window = 128
    E = idx_flat.shape[1]
    D = table.shape[1]
    n_workers = 32
    steps = E // (n_workers * window)
    mesh = plsc.VectorSubcoreMesh(core_axis_name="c", subcore_axis_name="s")

    @functools.partial(
        pl.kernel,
        out_type=jax.ShapeDtypeStruct((E, D), table.dtype),
        mesh=mesh,
    )
    def gather_kernel(tab_hbm, idx_hbm, out_hbm):
        def body(idx_vmem, out_vmem):
            pltpu.sync_copy(tab_hbm.at[idx_vmem.at[0]], out_vmem)

        pltpu.emit_pipeline(
            body,
            grid=(n_workers, steps),
            in_specs=[
                pl.BlockSpec((1, window), lambda w, i, _s=steps: (0, w * _s + i))
            ],
            out_specs=[
                pl.BlockSpec((window, D), lambda w, i, _s=steps: (w * _s + i, 0))
            ],
            core_axis_name=("c", "s"),
            dimension_semantics=(pltpu.PARALLEL, pltpu.ARBITRARY),
        )(idx_hbm, out_hbm)

    return gather_kernel(table, idx_flat)


def _dot(a, b):
    return jnp.dot(a, b, preferred_element_type=jnp.float32)


def _atom_body(M, G, atom_ref, gath_ref, nbrf_ref, nidx_ref, state_ref,
               w_self_ref, w_nbr_ref, w_bond_ref, w_state_ref, ba_ref,
               w_bself_ref, atom_out_ref, pself_ref, seg_ref):
    i = pl.program_id(0)
    Bn, A = atom_ref.shape
    A2 = ba_ref.shape[1]

    atom = atom_ref[...]
    gath = gath_ref[...]
    nbrf = nbrf_ref[...]
    nidx = nidx_ref[0, 0, :]

    onehot = (nidx[:, None] == lax.broadcasted_iota(jnp.int32, (Bn, G), 1)
              ).astype(jnp.float32)

    z_edge = _dot(gath, w_nbr_ref[...]) + _dot(nbrf, w_bond_ref[...])
    sproj = _dot(state_ref[...], w_state_ref[...])
    z_atom = _dot(atom, w_self_ref[...]) + _dot(onehot, sproj) + ba_ref[...]
    z = z_edge.reshape(Bn, M, A2) + z_atom[:, None, :]
    filt = z[..., :A]
    core = z[..., A:]
    gated = jax.nn.sigmoid(filt) * _softplus(core)
    gsum = jnp.sum(gated, axis=1)
    atom_out = _softplus(atom + gsum)
    atom_out_ref[...] = atom_out
    pself_ref[...] = _dot(atom_out, w_bself_ref[...])

    onehot_t = (lax.broadcasted_iota(jnp.int32, (G, Bn), 0) == nidx[None, :]
                ).astype(jnp.float32)

    @pl.when(i == 0)
    def _():
        seg_ref[...] = jnp.zeros_like(seg_ref)

    seg_ref[...] += _dot(onehot_t, atom_out)


def _bond_body(M, G, nsteps, nbrf_ref, gatha_ref, pself_ref, nidx_ref,
               state_ref, seg_atom_ref, w_nbr_ref, w_bond_ref, w_state_ref,
               bb_ref, ws_a_ref, ws_b_ref, ws_s_ref, bs_ref,
               nbr_out_ref, state_out_ref, segb_scr, cnt_scr):
    i = pl.program_id(0)
    Bn = pself_ref.shape[0]
    NB = nbrf_ref.shape[1]

    nbrf = nbrf_ref[...]
    gathp = _dot(gatha_ref[...], w_nbr_ref[...])
    nidx = nidx_ref[0, 0, :]

    onehot = (nidx[:, None] == lax.broadcasted_iota(jnp.int32, (Bn, G), 1)
              ).astype(jnp.float32)
    onehot_t = (lax.broadcasted_iota(jnp.int32, (G, Bn), 0) == nidx[None, :]
                ).astype(jnp.float32)

    zb_atom = (pself_ref[...] + _dot(onehot, _dot(state_ref[...], w_state_ref[...]))
               + bb_ref[...])
    zb_edge = _dot(nbrf, w_bond_ref[...]) + gathp
    nbr3 = nbrf.reshape(Bn, M, NB)
    zb = zb_edge.reshape(Bn, M, NB) + zb_atom[:, None, :]
    nbr_out = _softplus(nbr3 + zb)
    nbr_out_ref[...] = nbr_out.reshape(Bn * M, NB)
    bma = jnp.sum(nbr_out, axis=1) * (1.0 / M)

    @pl.when(i == 0)
    def _():
        segb_scr[...] = jnp.zeros_like(segb_scr)
        cnt_scr[...] = jnp.zeros_like(cnt_scr)

    segb_scr[...] += _dot(onehot_t, bma)
    cnt_scr[...] += _dot(onehot_t, jnp.ones((Bn, 128), jnp.float32))

    @pl.when(i == nsteps - 1)
    def _():
        cnt = jnp.maximum(cnt_scr[...], 1.0)
        atom_mean = seg_atom_ref[...] / cnt
        bond_mean = segb_scr[...] / cnt[:, :NB]
        st = state_ref[...]
        state_out_ref[...] = _softplus(
            st + _dot(atom_mean, ws_a_ref[...]) + _dot(bond_mean, ws_b_ref[...])
            + _dot(st, ws_s_ref[...]) + bs_ref[...])


def kernel(atom_fea, nbr_fea, state_fea, atom_nbr_idx, node_atom_idx,
           W_a, b_a, W_b, b_b, W_s, b_s):
    N, A = atom_fea.shape
    M = nbr_fea.shape[1]
    NB = nbr_fea.shape[2]
    G, S = state_fea.shape
    A2 = 2 * A
    E = N * M

    Bn = next(b for b in (200, 250, 100, 50, 40, 25, 20, 10, 8, 5, 4, 2, 1)
              if N % b == 0)
    nsteps = N // Bn

    Ep = ((E + 4095) // 4096) * 4096
    idx_flat = jnp.pad(atom_nbr_idx.reshape(1, E).astype(jnp.int32),
                       ((0, 0), (0, Ep - E)))
    nbrf_flat = nbr_fea.reshape(E, NB)
    nidx3 = node_atom_idx.astype(jnp.int32).reshape(nsteps, 1, Bn)

    wa_self = W_a[:A]
    wa_nbr = W_a[A:2 * A]
    wa_bond = W_a[2 * A:2 * A + NB]
    wa_state = W_a[2 * A + NB:]
    wb_self = W_b[:A]
    wb_nbr = W_b[A:2 * A]
    wb_bond = W_b[2 * A:2 * A + NB]
    wb_state = W_b[2 * A + NB:]
    ws_a = W_s[:A]
    ws_b = W_s[A:A + NB]
    ws_s = W_s[A + NB:]
    ba2 = b_a.reshape(1, A2)
    bb2 = b_b.reshape(1, NB)
    bs2 = b_s.reshape(1, S)

    gath1 = _sc_gather(atom_fea, idx_flat)

    atom_out, p_self, seg_atom = pl.pallas_call(
        functools.partial(_atom_body, M, G),
        grid=(nsteps,),
        in_specs=[
            pl.BlockSpec((Bn, A), lambda i: (i, 0)),
            pl.BlockSpec((Bn * M, A), lambda i: (i, 0)),
            pl.BlockSpec((Bn * M, NB), lambda i: (i, 0)),
            pl.BlockSpec((1, 1, Bn), lambda i: (i, 0, 0)),
            pl.BlockSpec((G, S), lambda i: (0, 0)),
            pl.BlockSpec((A, A2), lambda i: (0, 0)),
            pl.BlockSpec((A, A2), lambda i: (0, 0)),
            pl.BlockSpec((NB, A2), lambda i: (0, 0)),
            pl.BlockSpec((S, A2), lambda i: (0, 0)),
            pl.BlockSpec((1, A2), lambda i: (0, 0)),
            pl.BlockSpec((A, NB), lambda i: (0, 0)),
        ],
        out_specs=[
            pl.BlockSpec((Bn, A), lambda i: (i, 0)),
            pl.BlockSpec((Bn, NB), lambda i: (i, 0)),
            pl.BlockSpec((G, A), lambda i: (0, 0)),
        ],
        out_shape=[
            jax.ShapeDtypeStruct((N, A), jnp.float32),
            jax.ShapeDtypeStruct((N, NB), jnp.float32),
            jax.ShapeDtypeStruct((G, A), jnp.float32),
        ],
    )(atom_fea, gath1, nbrf_flat, nidx3, state_fea,
      wa_self, wa_nbr, wa_bond, wa_state, ba2, wb_self)

    gathp = _sc_gather(atom_out, idx_flat)

    nbr_out_flat, state_out = pl.pallas_call(
        functools.partial(_bond_body, M, G, nsteps),
        grid=(nsteps,),
        in_specs=[
            pl.BlockSpec((Bn * M, NB), lambda i: (i, 0)),
            pl.BlockSpec((Bn * M, A), lambda i: (i, 0)),
            pl.BlockSpec((Bn, NB), lambda i: (i, 0)),
            pl.BlockSpec((1, 1, Bn), lambda i: (i, 0, 0)),
            pl.BlockSpec((G, S), lambda i: (0, 0)),
            pl.BlockSpec((G, A), lambda i: (0, 0)),
            pl.BlockSpec((A, NB), lambda i: (0, 0)),
            pl.BlockSpec((NB, NB), lambda i: (0, 0)),
            pl.BlockSpec((S, NB), lambda i: (0, 0)),
            pl.BlockSpec((1, NB), lambda i: (0, 0)),
            pl.BlockSpec((A, S), lambda i: (0, 0)),
            pl.BlockSpec((NB, S), lambda i: (0, 0)),
            pl.BlockSpec((S, S), lambda i: (0, 0)),
            pl.BlockSpec((1, S), lambda i: (0, 0)),
        ],
        out_specs=[
            pl.BlockSpec((Bn * M, NB), lambda i: (i, 0)),
            pl.BlockSpec((G, S), lambda i: (0, 0)),
        ],
        out_shape=[
            jax.ShapeDtypeStruct((E, NB), jnp.float32),
            jax.ShapeDtypeStruct((G, S), jnp.float32),
        ],
        scratch_shapes=[
            pltpu.VMEM((G, NB), jnp.float32),
            pltpu.VMEM((G, 128), jnp.float32),
        ],
    )(nbrf_flat, gathp, p_self, nidx3, state_fea, seg_atom,
      wb_nbr, wb_bond, wb_state, bb2, ws_a, ws_b, ws_s, bs2)

    return atom_out, nbr_out_flat.reshape(N, M, NB), state_out
